# Initial kernel scaffold; baseline (speedup 1.0000x reference)
import jax
import jax.numpy as jnp
from jax.experimental import pallas as pl


def kernel(x, tables):
    raise NotImplementedError("write your pallas kernel here")


# speedup vs baseline: 12.6804x; 12.6804x over previous
import functools

import jax
import jax.numpy as jnp
from jax import lax
from jax.experimental import pallas as pl
from jax.experimental.pallas import tpu as pltpu
from jax.experimental.pallas import tpu_sc as plsc

F = 26
V = 51
D = 64
B = 16384
NC = 2
NS = 16
NW = NC * NS
ROWS = B * F
RPW = ROWS // NW
SUB = 128
NROW = RPW // SUB
CHUNK = 512
NSUB = CHUNK // SUB
NCHUNK = RPW // CHUNK

_mesh = plsc.VectorSubcoreMesh(
    core_axis_name="c", subcore_axis_name="s", num_cores=NC, num_subcores=NS)


@functools.partial(
    pl.kernel,
    out_type=jax.ShapeDtypeStruct((ROWS, D), jnp.float32),
    mesh=_mesh,
    scratch_types=[
        pltpu.VMEM((NROW, SUB), jnp.int32),
        pltpu.VMEM((NROW, SUB), jnp.int32),
        pltpu.VMEM((2, CHUNK, D), jnp.float32),
        pltpu.SemaphoreType.DMA,
        pltpu.SemaphoreType.DMA,
    ],
    compiler_params=pltpu.CompilerParams(use_tc_tiling_on_sc=False),
)
def _emb_gather(tab_hbm, idx_hbm, off_hbm, out_hbm,
                idx_v, off_v, rows_v, sem0, sem1):
    wid = lax.axis_index("s") * NC + lax.axis_index("c")

    pltpu.sync_copy(idx_hbm.at[wid], idx_v)
    pltpu.sync_copy(off_hbm, off_v)

    def _add_row(i):
        for k in range(SUB // 16):
            s = pl.ds(k * 16, 16)
            idx_v[i, s] = idx_v[i, s] + off_v[i, s]
    pl.loop(0, NROW)(_add_row)

    sems = (sem0, sem1)

    def start(c, slot):
        for j in range(NSUB):
            pltpu.async_copy(
                tab_hbm.at[idx_v.at[c * NSUB + j]],
                rows_v.at[slot, pl.ds(j * SUB, SUB)],
                sems[slot])

    def finish(c, slot):
        for j in range(NSUB):
            pltpu.make_async_copy(
                tab_hbm.at[idx_v.at[c * NSUB + j]],
                rows_v.at[slot, pl.ds(j * SUB, SUB)],
                sems[slot]).wait()
        pltpu.sync_copy(
            rows_v.at[slot],
            out_hbm.at[pl.ds((wid * NCHUNK + c) * CHUNK, CHUNK)])

    start(0, 0)

    def _pipe(c):
        start(c + 1, 1)
        finish(c, 0)
        start(c + 2, 0)
        finish(c + 1, 1)
    pl.loop(0, NCHUNK - 2, step=2)(_pipe)

    start(NCHUNK - 1, 1)
    finish(NCHUNK - 2, 0)
    finish(NCHUNK - 1, 1)


def kernel(x, tables):
    tab_flat = tables.reshape(F * V, D)
    idx3 = x.reshape(NW, NROW, SUB)
    off = jnp.tile(jnp.arange(F, dtype=jnp.int32) * V, RPW // F).reshape(NROW, SUB)
    out = _emb_gather(tab_flat, idx3, off)
    return out.reshape(B, F, D)
